# Initial kernel scaffold; baseline (speedup 1.0000x reference)
import jax
import jax.numpy as jnp
from jax.experimental import pallas as pl


def kernel(x, edge_index, y_edge_index, W_p, b_p, W1, b1, W_mu, b_mu, W_ls, b_ls):
    raise NotImplementedError("write your pallas kernel here")


# speedup vs baseline: 2.6717x; 2.6717x over previous
import functools

import jax
import jax.numpy as jnp
from jax.experimental import pallas as pl
from jax.experimental.pallas import tpu as pltpu


def _mm_scale(x, W, scale, BM=1000):
    M, K = x.shape
    _, Nt = W.shape

    def body(x_ref, w_ref, s_ref, o_ref):
        acc = jnp.dot(x_ref[...], w_ref[...], preferred_element_type=jnp.float32)
        o_ref[...] = acc * s_ref[...]

    return pl.pallas_call(
        body,
        grid=(M // BM,),
        in_specs=[
            pl.BlockSpec((BM, K), lambda i: (i, 0)),
            pl.BlockSpec((K, Nt), lambda i: (0, 0)),
            pl.BlockSpec((BM, 1), lambda i: (i, 0)),
        ],
        out_specs=pl.BlockSpec((BM, Nt), lambda i: (i, 0)),
        out_shape=jax.ShapeDtypeStruct((M, Nt), jnp.float32),
    )(x, W, scale)


def _relu_scale_mm_scale(s1, u, b, W, BM=1000):
    M, K = s1.shape
    _, Nt = W.shape

    def body(s_ref, u_ref, b_ref, w_ref, o_ref):
        h = jnp.maximum(u_ref[...] * s_ref[...] + b_ref[...], 0.0)
        acc = jnp.dot(h, w_ref[...], preferred_element_type=jnp.float32)
        o_ref[...] = acc * u_ref[...]

    return pl.pallas_call(
        body,
        grid=(M // BM,),
        in_specs=[
            pl.BlockSpec((BM, K), lambda i: (i, 0)),
            pl.BlockSpec((BM, 1), lambda i: (i, 0)),
            pl.BlockSpec((1, K), lambda i: (0, 0)),
            pl.BlockSpec((K, Nt), lambda i: (0, 0)),
        ],
        out_specs=pl.BlockSpec((BM, Nt), lambda i: (i, 0)),
        out_shape=jax.ShapeDtypeStruct((M, Nt), jnp.float32),
    )(s1, u, b.reshape(1, K), W)


def kernel(x, edge_index, y_edge_index, W_p, b_p, W1, b1, W_mu, b_mu, W_ls, b_ls):
    N = x.shape[0]
    src = jnp.concatenate([edge_index[0], y_edge_index[0]])
    dst = jnp.concatenate([edge_index[1], y_edge_index[1]])

    deg = 1.0 + jnp.zeros((N,), jnp.float32).at[dst].add(1.0)
    u = jax.lax.rsqrt(deg).reshape(N, 1)

    g1 = _mm_scale(x, W1, u)
    s1 = g1 + jnp.zeros_like(g1).at[dst].add(g1[src])
    g2 = _relu_scale_mm_scale(s1, u, b1, W_mu)
    s2 = g2 + jnp.zeros_like(g2).at[dst].add(g2[src])
    out = u * s2 + b_mu
    return out
